# Initial kernel scaffold; baseline (speedup 1.0000x reference)
import jax
import jax.numpy as jnp
from jax.experimental import pallas as pl


def kernel(x, edge_index, weight, batch, W_l, b_l, W_r, b_r, att, bias):
    raise NotImplementedError("write your pallas kernel here")


# speedup vs baseline: 4.0576x; 4.0576x over previous
import functools

import jax
import jax.numpy as jnp
from jax import lax
from jax.experimental import pallas as pl
from jax.experimental.pallas import tpu as pltpu
from jax.experimental.pallas import tpu_sc as plsc

D = 128
NW = 32
NTILES = 16
CHUNK = 128


def _proj_body(x_ref, wl_ref, wr_ref, bl_ref, br_ref, xl_ref, xr_ref):
    xb = x_ref[...]
    xl_ref[...] = jnp.dot(xb, wl_ref[...],
                          preferred_element_type=jnp.float32) + bl_ref[...]
    xr_ref[...] = jnp.dot(xb, wr_ref[...],
                          preferred_element_type=jnp.float32) + br_ref[...]


def _norm_body(acc_ref, den_ref, bias_ref, out_ref):
    a = acc_ref[...]
    s = a[0] + a[1]
    den = jnp.sum(den_ref[...], axis=0)
    out_ref[...] = s / (den[:, None] + 1e-16) + bias_ref[...]


def _make_edge_kernel(np_, ep):
    per_w = ep // NW
    n_chunks = per_w // CHUNK
    rows_per_tile = np_ // NTILES
    mesh = plsc.VectorSubcoreMesh(core_axis_name="c", subcore_axis_name="s")

    @functools.partial(
        pl.kernel,
        mesh=mesh,
        out_type=[
            jax.ShapeDtypeStruct((2, np_, D), jnp.float32),
            jax.ShapeDtypeStruct((NW, np_), jnp.float32),
        ],
        scratch_types=[
            pltpu.VMEM((CHUNK,), jnp.int32),
            pltpu.VMEM((CHUNK,), jnp.int32),
            pltpu.VMEM((CHUNK, D), jnp.float32),
            pltpu.VMEM((CHUNK, D), jnp.float32),
            pltpu.VMEM((D,), jnp.float32),
            pltpu.VMEM((np_,), jnp.float32),
            pltpu.VMEM_SHARED((np_, D), jnp.float32),
            pltpu.SemaphoreType.DMA,
            pltpu.SemaphoreType.DMA,
        ],
        compiler_params=pltpu.CompilerParams(needs_layout_passes=False),
    )
    def edge_kernel(xl_hbm, xr_hbm, att_hbm, src_hbm, dst_hbm, zero_hbm,
                    out_hbm, den_hbm,
                    src_v, dst_v, xl_v, xr_v, att_v, den_v, acc_sh,
                    sem1, sem2):
        cid = lax.axis_index("c")
        sid = lax.axis_index("s")
        wid = sid * jnp.int32(2) + cid
        row0 = sid * jnp.int32(rows_per_tile)
        pltpu.sync_copy(zero_hbm.at[pl.ds(row0, rows_per_tile)],
                        acc_sh.at[pl.ds(row0, rows_per_tile)])
        pltpu.sync_copy(att_hbm, att_v)

        zvec = jnp.zeros((16,), jnp.float32)

        def zero_den(i, c):
            den_v[pl.ds(i * jnp.int32(16), 16)] = zvec
            return c

        lax.fori_loop(jnp.int32(0), jnp.int32(np_ // 16), zero_den, jnp.int32(0))
        plsc.subcore_barrier()
        att_s = [att_v[pl.ds(16 * j, 16)] for j in range(D // 16)]
        lane = lax.iota(jnp.int32, 16)

        def chunk_body(g, carry):
            base = wid * jnp.int32(per_w) + g * jnp.int32(CHUNK)
            pltpu.sync_copy(src_hbm.at[pl.ds(base, CHUNK)], src_v)
            pltpu.sync_copy(dst_hbm.at[pl.ds(base, CHUNK)], dst_v)
            cp1 = pltpu.async_copy(xl_hbm.at[src_v], xl_v, sem1)
            cp2 = pltpu.async_copy(xr_hbm.at[dst_v], xr_v, sem2)
            cp1.wait()
            cp2.wait()

            def grp_logit(gg, c):
                rows = lane + gg * jnp.int32(16)
                acc = jnp.zeros((16,), jnp.float32)
                for k in range(D):
                    kv = jnp.full((16,), k, jnp.int32)
                    a = plsc.load_gather(xl_v, [rows, kv])
                    b = plsc.load_gather(xr_v, [rows, kv])
                    v = a + b
                    v = jnp.maximum(v, 0.2 * v)
                    acc = acc + att_s[k // 16][k % 16] * v
                pvec = jnp.exp(acc)
                dst_g = dst_v[pl.ds(gg * jnp.int32(16), 16)]
                for t in range(16):
                    plsc.addupdate_scatter(den_v, [dst_g], pvec,
                                           mask=lane == jnp.int32(t))
                for t in range(16):
                    e = gg * jnp.int32(16) + jnp.int32(t)
                    p = pvec[t]
                    for j in range(D // 16):
                        sl = pl.ds(16 * j, 16)
                        xl_v[e, sl] = xl_v[e, sl] * p
                return c

            lax.fori_loop(jnp.int32(0), jnp.int32(CHUNK // 16), grp_logit, jnp.int32(0))
            pltpu.sync_copy(xl_v, acc_sh.at[dst_v], add=True)
            return carry

        lax.fori_loop(jnp.int32(0), jnp.int32(n_chunks), chunk_body, jnp.int32(0))
        plsc.subcore_barrier()
        pltpu.sync_copy(acc_sh.at[pl.ds(row0, rows_per_tile)],
                        out_hbm.at[cid, pl.ds(row0, rows_per_tile)])
        pltpu.sync_copy(den_v, den_hbm.at[wid])

    return edge_kernel


def kernel(x, edge_index, weight, batch, W_l, b_l, W_r, b_r, att, bias):
    n = x.shape[0]
    e = edge_index.shape[1]
    np_ = ((n + 16 + 255) // 256) * 256
    etot = e + n
    ep = ((etot + NW * CHUNK - 1) // (NW * CHUNK)) * (NW * CHUNK)

    loop = jnp.arange(n, dtype=jnp.int32)
    src = jnp.concatenate([
        edge_index[0].astype(jnp.int32), loop,
        jnp.zeros((ep - etot,), jnp.int32)])
    dst = jnp.concatenate([
        edge_index[1].astype(jnp.int32), loop,
        jnp.full((ep - etot,), n, jnp.int32)])

    x_pad = jnp.pad(x, ((0, np_ - n), (0, 0)))

    rows_blk = 256
    grid1 = np_ // rows_blk
    xl_pad, xr_pad = pl.pallas_call(
        _proj_body,
        grid=(grid1,),
        in_specs=[
            pl.BlockSpec((rows_blk, D), lambda i: (i, jnp.int32(0))),
            pl.BlockSpec((D, D), lambda i: (jnp.int32(0), jnp.int32(0))),
            pl.BlockSpec((D, D), lambda i: (jnp.int32(0), jnp.int32(0))),
            pl.BlockSpec((1, D), lambda i: (jnp.int32(0), jnp.int32(0))),
            pl.BlockSpec((1, D), lambda i: (jnp.int32(0), jnp.int32(0))),
        ],
        out_specs=[
            pl.BlockSpec((rows_blk, D), lambda i: (i, jnp.int32(0))),
            pl.BlockSpec((rows_blk, D), lambda i: (i, jnp.int32(0))),
        ],
        out_shape=[
            jax.ShapeDtypeStruct((np_, D), jnp.float32),
            jax.ShapeDtypeStruct((np_, D), jnp.float32),
        ],
    )(x_pad, W_l, W_r, b_l.reshape(1, D), b_r.reshape(1, D))

    zero = jnp.zeros((np_, D), jnp.float32)
    acc, den = _make_edge_kernel(np_, ep)(
        xl_pad, xr_pad, att.astype(jnp.float32), src, dst, zero)

    out_blk = 512
    grid3 = np_ // out_blk
    out = pl.pallas_call(
        _norm_body,
        grid=(grid3,),
        in_specs=[
            pl.BlockSpec((2, out_blk, D), lambda i: (jnp.int32(0), i, jnp.int32(0))),
            pl.BlockSpec((NW, out_blk), lambda i: (jnp.int32(0), i)),
            pl.BlockSpec((1, D), lambda i: (jnp.int32(0), jnp.int32(0))),
        ],
        out_specs=pl.BlockSpec((out_blk, D), lambda i: (i, jnp.int32(0))),
        out_shape=jax.ShapeDtypeStruct((np_, D), jnp.float32),
    )(acc, den, bias.reshape(1, D))
    return out[:n]
